# baseline (speedup 1.0000x reference)
import functools

import jax
import jax.numpy as jnp
from jax import lax
from jax.experimental import pallas as pl
from jax.experimental.pallas import tpu as pltpu
from jax.experimental.pallas import tpu_sc as plsc


@functools.cache
def _build(B, D, C):
    info = plsc.get_sparse_core_info()
    NC, NS = info.num_cores, info.num_subcores
    NW = NC * NS
    b_per_w = B // NW
    n_chunks = b_per_w // C
    assert b_per_w * NW == B and n_chunks * C == b_per_w

    mesh = plsc.VectorSubcoreMesh(core_axis_name="c", subcore_axis_name="s")

    @functools.partial(
        pl.kernel,
        mesh=mesh,
        out_type=jax.ShapeDtypeStruct((B, D), jnp.float32),
        scratch_types=[
            pltpu.VMEM((C,), jnp.int32),
            pltpu.VMEM((C, D), jnp.float32),
            pltpu.SemaphoreType.DMA,
        ],
        compiler_params=pltpu.CompilerParams(use_tc_tiling_on_sc=False),
    )
    def gather_kernel(idx_hbm, table_hbm, out_hbm, idx_v, rows_v, sem):
        wid = lax.axis_index("s") * NC + lax.axis_index("c")
        base = wid * b_per_w

        @pl.loop(0, n_chunks)
        def _(i):
            off = base + i * C
            pltpu.sync_copy(idx_hbm.at[pl.ds(off, C)], idx_v)
            pltpu.async_copy(table_hbm.at[idx_v], rows_v, sem).wait()
            pltpu.sync_copy(rows_v, out_hbm.at[pl.ds(off, C)])

    return gather_kernel


def kernel(x, word_vectors):
    B = x.shape[0] * x.shape[1]
    D = word_vectors.shape[1]
    idx = x.reshape(B)
    out = _build(B, D, 512)(idx, word_vectors)
    return out.reshape(x.shape[0], x.shape[1], D)


# speedup vs baseline: 1.0382x; 1.0382x over previous
import functools

import jax
import jax.numpy as jnp
from jax import lax
from jax.experimental import pallas as pl
from jax.experimental.pallas import tpu as pltpu
from jax.experimental.pallas import tpu_sc as plsc


@functools.cache
def _build(B, D, C):
    info = plsc.get_sparse_core_info()
    NC, NS = info.num_cores, info.num_subcores
    NW = NC * NS
    b_per_w = B // NW
    n_chunks = b_per_w // C
    assert b_per_w * NW == B and n_chunks * C == b_per_w and n_chunks % 2 == 0

    mesh = plsc.VectorSubcoreMesh(core_axis_name="c", subcore_axis_name="s")

    @functools.partial(
        pl.kernel,
        mesh=mesh,
        out_type=jax.ShapeDtypeStruct((B, D), jnp.float32),
        scratch_types=[
            pltpu.VMEM((b_per_w,), jnp.int32),
            pltpu.VMEM((C, D), jnp.float32),
            pltpu.VMEM((C, D), jnp.float32),
            pltpu.SemaphoreType.DMA,
            pltpu.SemaphoreType.DMA,
            pltpu.SemaphoreType.DMA,
            pltpu.SemaphoreType.DMA,
        ],
        compiler_params=pltpu.CompilerParams(use_tc_tiling_on_sc=False),
    )
    def gather_kernel(idx_hbm, table_hbm, out_hbm, idx_v, buf0, buf1,
                      g0, g1, s0, s1):
        wid = lax.axis_index("s") * NC + lax.axis_index("c")
        base = wid * b_per_w
        pltpu.sync_copy(idx_hbm.at[pl.ds(base, b_per_w)], idx_v)

        @pl.loop(0, n_chunks, step=2)
        def _(i):
            ga = pltpu.async_copy(
                table_hbm.at[idx_v.at[pl.ds(i * C, C)]], buf0, g0)
            gb = pltpu.async_copy(
                table_hbm.at[idx_v.at[pl.ds((i + 1) * C, C)]], buf1, g1)
            ga.wait()
            sa = pltpu.async_copy(buf0, out_hbm.at[pl.ds(base + i * C, C)], s0)
            gb.wait()
            sb = pltpu.async_copy(
                buf1, out_hbm.at[pl.ds(base + (i + 1) * C, C)], s1)
            sa.wait()
            sb.wait()

    return gather_kernel


def kernel(x, word_vectors):
    B = x.shape[0] * x.shape[1]
    D = word_vectors.shape[1]
    idx = x.reshape(B)
    out = _build(B, D, 640)(idx, word_vectors)
    return out.reshape(x.shape[0], x.shape[1], D)


# speedup vs baseline: 1.2562x; 1.2100x over previous
import functools

import jax
import jax.numpy as jnp
from jax import lax
from jax.experimental import pallas as pl
from jax.experimental.pallas import tpu as pltpu
from jax.experimental.pallas import tpu_sc as plsc


@functools.cache
def _build(B, DP, C):
    info = plsc.get_sparse_core_info()
    NC, NS = info.num_cores, info.num_subcores
    NW = NC * NS
    b_per_w = B // NW
    n_chunks = b_per_w // C
    assert b_per_w * NW == B and n_chunks * C == b_per_w and n_chunks % 2 == 0

    mesh = plsc.VectorSubcoreMesh(core_axis_name="c", subcore_axis_name="s")

    @functools.partial(
        pl.kernel,
        mesh=mesh,
        out_type=jax.ShapeDtypeStruct((B, DP), jnp.float32),
        scratch_types=[
            pltpu.VMEM((b_per_w,), jnp.int32),
            pltpu.VMEM((C, DP), jnp.float32),
            pltpu.VMEM((C, DP), jnp.float32),
            pltpu.SemaphoreType.DMA,
            pltpu.SemaphoreType.DMA,
            pltpu.SemaphoreType.DMA,
            pltpu.SemaphoreType.DMA,
        ],
        compiler_params=pltpu.CompilerParams(use_tc_tiling_on_sc=False),
    )
    def gather_kernel(idx_hbm, table_hbm, out_hbm, idx_v, buf0, buf1,
                      g0, g1, s0, s1):
        wid = lax.axis_index("s") * NC + lax.axis_index("c")
        base = wid * b_per_w
        pltpu.sync_copy(idx_hbm.at[pl.ds(base, b_per_w)], idx_v)

        @pl.loop(0, n_chunks, step=2)
        def _(i):
            ga = pltpu.async_copy(
                table_hbm.at[idx_v.at[pl.ds(i * C, C)]], buf0, g0)
            gb = pltpu.async_copy(
                table_hbm.at[idx_v.at[pl.ds((i + 1) * C, C)]], buf1, g1)
            ga.wait()
            sa = pltpu.async_copy(buf0, out_hbm.at[pl.ds(base + i * C, C)], s0)
            gb.wait()
            sb = pltpu.async_copy(
                buf1, out_hbm.at[pl.ds(base + (i + 1) * C, C)], s1)
            sa.wait()
            sb.wait()

    return gather_kernel


def kernel(x, word_vectors):
    B = x.shape[0] * x.shape[1]
    D = word_vectors.shape[1]
    DP = 128
    idx = x.reshape(B)
    wv_pad = jnp.pad(word_vectors, ((0, 0), (0, DP - D)))
    out_pad = _build(B, DP, 256)(idx, wv_pad)
    return out_pad[:, :D].reshape(x.shape[0], x.shape[1], D)


# speedup vs baseline: 1.3778x; 1.0968x over previous
import functools

import jax
import jax.numpy as jnp
from jax import lax
from jax.experimental import pallas as pl
from jax.experimental.pallas import tpu as pltpu
from jax.experimental.pallas import tpu_sc as plsc


@functools.cache
def _build(B, D, DP, C):
    info = plsc.get_sparse_core_info()
    NC, NS = info.num_cores, info.num_subcores
    NW = NC * NS
    b_per_w = B // NW
    n_chunks = b_per_w // C
    assert b_per_w * NW == B and n_chunks * C == b_per_w and n_chunks % 2 == 0

    mesh = plsc.VectorSubcoreMesh(core_axis_name="c", subcore_axis_name="s")

    @functools.partial(
        pl.kernel,
        mesh=mesh,
        out_type=jax.ShapeDtypeStruct((B, DP), jnp.float32),
        scratch_types=[
            pltpu.VMEM((b_per_w,), jnp.int32),
            pltpu.VMEM((C, D), jnp.float32),
            pltpu.VMEM((C, D), jnp.float32),
            pltpu.SemaphoreType.DMA,
            pltpu.SemaphoreType.DMA,
            pltpu.SemaphoreType.DMA,
            pltpu.SemaphoreType.DMA,
        ],
        compiler_params=pltpu.CompilerParams(use_tc_tiling_on_sc=False),
    )
    def gather_kernel(idx_hbm, table_hbm, out_hbm, idx_v, buf0, buf1,
                      g0, g1, s0, s1):
        wid = lax.axis_index("s") * NC + lax.axis_index("c")
        base = wid * b_per_w
        pltpu.sync_copy(idx_hbm.at[pl.ds(base, b_per_w)], idx_v)

        @pl.loop(0, n_chunks, step=2)
        def _(i):
            ga = pltpu.async_copy(
                table_hbm.at[idx_v.at[pl.ds(i * C, C)]], buf0, g0)
            gb = pltpu.async_copy(
                table_hbm.at[idx_v.at[pl.ds((i + 1) * C, C)]], buf1, g1)
            ga.wait()
            sa = pltpu.async_copy(
                buf0, out_hbm.at[pl.ds(base + i * C, C), pl.ds(0, D)], s0)
            gb.wait()
            sb = pltpu.async_copy(
                buf1, out_hbm.at[pl.ds(base + (i + 1) * C, C), pl.ds(0, D)],
                s1)
            sa.wait()
            sb.wait()

    return gather_kernel


def kernel(x, word_vectors):
    B = x.shape[0] * x.shape[1]
    D = word_vectors.shape[1]
    DP = 128
    idx = x.reshape(B)
    wv_wide = lax.optimization_barrier(
        word_vectors.reshape(word_vectors.shape[0] // 2, 2 * D))
    wv_lin = wv_wide.reshape(word_vectors.shape[0], D)
    out_pad = _build(B, D, DP, 512)(idx, wv_lin)
    return out_pad[:, :D].reshape(x.shape[0], x.shape[1], D)
